# baseline (speedup 1.0000x reference)
import jax
import jax.numpy as jnp
from jax.experimental import pallas as pl
from jax.experimental.pallas import tpu as pltpu

_N = 10000
_FEAT = 128
_NHID = 256

_TMC = 80
_NC = _N // _TMC
_NBUF = 8
_TSUP = 2000
_CPS = _TSUP // _TMC
_NSUP = _N // _TSUP


def _outer(adj_hbm, x_ref, xbf_ref, wt_ref, wb_ref, b_ref, w1_ref, b1_ref,
           alpha_ref, w2_ref, b2_ref, out_ref, agg_ref):

    def _chunk(adj_blk):
        i = pl.program_id(0)
        agg_ref[pl.ds(i * _TMC, _TMC), :] = jnp.dot(
            adj_blk[...].astype(jnp.bfloat16), xbf_ref[...],
            preferred_element_type=jnp.float32)

        @pl.when(i % _CPS == _CPS - 1)
        def _epilogue():
            s = (i // _CPS) * _TSUP
            xm = x_ref[pl.ds(s, _TSUP), :]
            agg = agg_ref[pl.ds(s, _TSUP), :]
            h = jnp.dot(xm, wt_ref[...], preferred_element_type=jnp.float32)
            h += jnp.dot(agg, wb_ref[...], preferred_element_type=jnp.float32)
            h = jnp.maximum(h + b_ref[...], 0.0)
            h1 = jnp.dot(h, w1_ref[...], preferred_element_type=jnp.float32)
            h1 += b1_ref[...]
            h1 = jnp.where(h1 >= 0, h1, alpha_ref[...] * h1)
            out_ref[pl.ds(s, _TSUP), :] = b2_ref[0, 0] + jnp.dot(
                h1, w2_ref[...], preferred_element_type=jnp.float32)

    pipe = pltpu.emit_pipeline(
        _chunk,
        grid=(_NC,),
        in_specs=[
            pl.BlockSpec((_TMC, _N), lambda i: (i, 0),
                         pipeline_mode=pl.Buffered(buffer_count=_NBUF)),
        ],
    )
    pipe(adj_hbm)


def kernel(x, adj, W, b, W1, b1, alpha, W2, b2):
    wt = W[:_FEAT]
    wb = W[_FEAT:]
    w2p = jnp.pad(W2, ((0, 0), (0, 7)))
    out = pl.pallas_call(
        _outer,
        grid=(1,),
        in_specs=[
            pl.BlockSpec(memory_space=pltpu.MemorySpace.HBM),
            pl.BlockSpec((_N, _FEAT), lambda i: (0, 0)),
            pl.BlockSpec((_N, _FEAT), lambda i: (0, 0)),
            pl.BlockSpec((_FEAT, _NHID), lambda i: (0, 0)),
            pl.BlockSpec((_FEAT, _NHID), lambda i: (0, 0)),
            pl.BlockSpec((1, _NHID), lambda i: (0, 0)),
            pl.BlockSpec((_NHID, _NHID), lambda i: (0, 0)),
            pl.BlockSpec((1, _NHID), lambda i: (0, 0)),
            pl.BlockSpec((1, _NHID), lambda i: (0, 0)),
            pl.BlockSpec((_NHID, 8), lambda i: (0, 0)),
            pl.BlockSpec((1, 1), lambda i: (0, 0)),
        ],
        out_specs=pl.BlockSpec((_N, 8), lambda i: (0, 0)),
        out_shape=jax.ShapeDtypeStruct((_N, 8), jnp.float32),
        scratch_shapes=[pltpu.VMEM((_N, _FEAT), jnp.float32)],
        compiler_params=pltpu.CompilerParams(
            dimension_semantics=("arbitrary",),
        ),
    )(adj, x, x.astype(jnp.bfloat16), wt, wb, b.reshape(1, _NHID), W1,
      b1.reshape(1, _NHID), alpha.reshape(1, _NHID), w2p, b2.reshape(1, 1))
    return out[:, 0]


# speedup vs baseline: 1.0606x; 1.0606x over previous
import jax
import jax.numpy as jnp
from jax.experimental import pallas as pl
from jax.experimental.pallas import tpu as pltpu

_N = 10000
_FEAT = 128
_NHID = 256

_TMC = 80
_NC = _N // _TMC
_NBUF = 8
_TSUP = 2000
_CPS = _TSUP // _TMC
_NSUP = _N // _TSUP


def _outer(adj_hbm, x_ref, wt_ref, wb_ref, b_ref, w1_ref, b1_ref,
           alpha_ref, w2_ref, b2_ref, out_ref, agg_ref):

    def _chunk(adj_blk):
        i = pl.program_id(0)
        agg_ref[pl.ds(i * _TMC, _TMC), :] = jnp.dot(
            adj_blk[...], x_ref[...], preferred_element_type=jnp.float32)

        @pl.when(i % _CPS == _CPS - 1)
        def _epilogue():
            s = (i // _CPS) * _TSUP
            xm = x_ref[pl.ds(s, _TSUP), :]
            agg = agg_ref[pl.ds(s, _TSUP), :]
            h = jnp.dot(xm, wt_ref[...], preferred_element_type=jnp.float32)
            h += jnp.dot(agg, wb_ref[...], preferred_element_type=jnp.float32)
            h = jnp.maximum(h + b_ref[...], 0.0)
            h1 = jnp.dot(h, w1_ref[...], preferred_element_type=jnp.float32)
            h1 += b1_ref[...]
            h1 = jnp.where(h1 >= 0, h1, alpha_ref[...] * h1)
            pred = jnp.sum(h1 * w2_ref[...], axis=1) + b2_ref[0, 0]
            out_ref[i // _CPS, :] = pred

    pipe = pltpu.emit_pipeline(
        _chunk,
        grid=(_NC,),
        in_specs=[
            pl.BlockSpec((_TMC, _N), lambda i: (i, 0),
                         pipeline_mode=pl.Buffered(buffer_count=_NBUF)),
        ],
    )
    pipe(adj_hbm)


def kernel(x, adj, W, b, W1, b1, alpha, W2, b2):
    wt = W[:_FEAT]
    wb = W[_FEAT:]
    out = pl.pallas_call(
        _outer,
        grid=(1,),
        in_specs=[
            pl.BlockSpec(memory_space=pltpu.MemorySpace.HBM),
            pl.BlockSpec((_N, _FEAT), lambda i: (0, 0)),
            pl.BlockSpec((_FEAT, _NHID), lambda i: (0, 0)),
            pl.BlockSpec((_FEAT, _NHID), lambda i: (0, 0)),
            pl.BlockSpec((1, _NHID), lambda i: (0, 0)),
            pl.BlockSpec((_NHID, _NHID), lambda i: (0, 0)),
            pl.BlockSpec((1, _NHID), lambda i: (0, 0)),
            pl.BlockSpec((1, _NHID), lambda i: (0, 0)),
            pl.BlockSpec((1, _NHID), lambda i: (0, 0)),
            pl.BlockSpec((1, 1), lambda i: (0, 0)),
        ],
        out_specs=pl.BlockSpec((_NSUP, _TSUP), lambda i: (0, 0)),
        out_shape=jax.ShapeDtypeStruct((_NSUP, _TSUP), jnp.float32),
        scratch_shapes=[pltpu.VMEM((_N, _FEAT), jnp.float32)],
        compiler_params=pltpu.CompilerParams(
            dimension_semantics=("arbitrary",),
        ),
    )(adj, x, wt, wb, b.reshape(1, _NHID), W1, b1.reshape(1, _NHID),
      alpha.reshape(1, _NHID), W2.reshape(1, _NHID), b2.reshape(1, 1))
    return out.reshape(-1)


# speedup vs baseline: 1.0615x; 1.0009x over previous
import jax
import jax.numpy as jnp
from jax.experimental import pallas as pl
from jax.experimental.pallas import tpu as pltpu

_N = 10000
_FEAT = 128
_NHID = 256

_TMC = 80
_NC = _N // _TMC
_NBUF = 12
_TSUP = 2000
_CPS = _TSUP // _TMC
_NSUP = _N // _TSUP


def _outer(adj_hbm, x_ref, wt_ref, wb_ref, b_ref, w1_ref, b1_ref,
           alpha_ref, w2_ref, b2_ref, out_ref, agg_ref):

    def _chunk(adj_blk):
        i = pl.program_id(0)
        agg_ref[pl.ds(i * _TMC, _TMC), :] = jnp.dot(
            adj_blk[...], x_ref[...], preferred_element_type=jnp.float32)

        @pl.when(i % _CPS == _CPS - 1)
        def _epilogue():
            s = (i // _CPS) * _TSUP
            xm = x_ref[pl.ds(s, _TSUP), :]
            agg = agg_ref[pl.ds(s, _TSUP), :]
            h = jnp.dot(xm, wt_ref[...], preferred_element_type=jnp.float32)
            h += jnp.dot(agg, wb_ref[...], preferred_element_type=jnp.float32)
            h = jnp.maximum(h + b_ref[...], 0.0)
            h1 = jnp.dot(h, w1_ref[...], preferred_element_type=jnp.float32)
            h1 += b1_ref[...]
            h1 = jnp.where(h1 >= 0, h1, alpha_ref[...] * h1)
            pred = jnp.sum(h1 * w2_ref[...], axis=1) + b2_ref[0, 0]
            out_ref[i // _CPS, :] = pred

    pipe = pltpu.emit_pipeline(
        _chunk,
        grid=(_NC,),
        in_specs=[
            pl.BlockSpec((_TMC, _N), lambda i: (i, 0),
                         pipeline_mode=pl.Buffered(buffer_count=_NBUF)),
        ],
    )
    pipe(adj_hbm)


def kernel(x, adj, W, b, W1, b1, alpha, W2, b2):
    wt = W[:_FEAT]
    wb = W[_FEAT:]
    out = pl.pallas_call(
        _outer,
        grid=(1,),
        in_specs=[
            pl.BlockSpec(memory_space=pltpu.MemorySpace.HBM),
            pl.BlockSpec((_N, _FEAT), lambda i: (0, 0)),
            pl.BlockSpec((_FEAT, _NHID), lambda i: (0, 0)),
            pl.BlockSpec((_FEAT, _NHID), lambda i: (0, 0)),
            pl.BlockSpec((1, _NHID), lambda i: (0, 0)),
            pl.BlockSpec((_NHID, _NHID), lambda i: (0, 0)),
            pl.BlockSpec((1, _NHID), lambda i: (0, 0)),
            pl.BlockSpec((1, _NHID), lambda i: (0, 0)),
            pl.BlockSpec((1, _NHID), lambda i: (0, 0)),
            pl.BlockSpec((1, 1), lambda i: (0, 0)),
        ],
        out_specs=pl.BlockSpec((_NSUP, _TSUP), lambda i: (0, 0)),
        out_shape=jax.ShapeDtypeStruct((_NSUP, _TSUP), jnp.float32),
        scratch_shapes=[pltpu.VMEM((_N, _FEAT), jnp.float32)],
        compiler_params=pltpu.CompilerParams(
            dimension_semantics=("arbitrary",),
        ),
    )(adj, x, wt, wb, b.reshape(1, _NHID), W1, b1.reshape(1, _NHID),
      alpha.reshape(1, _NHID), W2.reshape(1, _NHID), b2.reshape(1, 1))
    return out.reshape(-1)


# speedup vs baseline: 1.0769x; 1.0145x over previous
import jax
import jax.numpy as jnp
from jax.experimental import pallas as pl
from jax.experimental.pallas import tpu as pltpu

_N = 10000
_FEAT = 128
_NHID = 256

_TMC = 80
_NC = _N // _TMC
_NBUF = 5
_TSUP = 2000
_CPS = _TSUP // _TMC
_NSUP = _N // _TSUP


def _outer(adj_hbm, x_ref, wt_ref, wb_ref, b_ref, w1_ref, b1_ref,
           alpha_ref, w2_ref, b2_ref, out_ref, agg_ref):

    def _chunk(adj_blk0, adj_blk1):
        i = pl.program_id(0)
        agg_ref[pl.ds(2 * i * _TMC, _TMC), :] = jnp.dot(
            adj_blk0[...], x_ref[...], preferred_element_type=jnp.float32)
        agg_ref[pl.ds((2 * i + 1) * _TMC, _TMC), :] = jnp.dot(
            adj_blk1[...], x_ref[...], preferred_element_type=jnp.float32)

        @pl.when(i % (_CPS // 2) == _CPS // 2 - 1)
        def _epilogue():
            s = (i // (_CPS // 2)) * _TSUP
            xm = x_ref[pl.ds(s, _TSUP), :]
            agg = agg_ref[pl.ds(s, _TSUP), :]
            h = jnp.dot(xm, wt_ref[...], preferred_element_type=jnp.float32)
            h += jnp.dot(agg, wb_ref[...], preferred_element_type=jnp.float32)
            h = jnp.maximum(h + b_ref[...], 0.0)
            h1 = jnp.dot(h, w1_ref[...], preferred_element_type=jnp.float32)
            h1 += b1_ref[...]
            h1 = jnp.where(h1 >= 0, h1, alpha_ref[...] * h1)
            pred = jnp.sum(h1 * w2_ref[...], axis=1) + b2_ref[0, 0]
            out_ref[i // (_CPS // 2), :] = pred

    pipe = pltpu.emit_pipeline(
        _chunk,
        grid=(_NC // 2,),
        in_specs=[
            pl.BlockSpec((_TMC, _N), lambda i: (2 * i, 0),
                         pipeline_mode=pl.Buffered(buffer_count=_NBUF)),
            pl.BlockSpec((_TMC, _N), lambda i: (2 * i + 1, 0),
                         pipeline_mode=pl.Buffered(buffer_count=_NBUF)),
        ],
    )
    pipe(adj_hbm, adj_hbm)


def kernel(x, adj, W, b, W1, b1, alpha, W2, b2):
    wt = W[:_FEAT]
    wb = W[_FEAT:]
    out = pl.pallas_call(
        _outer,
        grid=(1,),
        in_specs=[
            pl.BlockSpec(memory_space=pltpu.MemorySpace.HBM),
            pl.BlockSpec((_N, _FEAT), lambda i: (0, 0)),
            pl.BlockSpec((_FEAT, _NHID), lambda i: (0, 0)),
            pl.BlockSpec((_FEAT, _NHID), lambda i: (0, 0)),
            pl.BlockSpec((1, _NHID), lambda i: (0, 0)),
            pl.BlockSpec((_NHID, _NHID), lambda i: (0, 0)),
            pl.BlockSpec((1, _NHID), lambda i: (0, 0)),
            pl.BlockSpec((1, _NHID), lambda i: (0, 0)),
            pl.BlockSpec((1, _NHID), lambda i: (0, 0)),
            pl.BlockSpec((1, 1), lambda i: (0, 0)),
        ],
        out_specs=pl.BlockSpec((_NSUP, _TSUP), lambda i: (0, 0)),
        out_shape=jax.ShapeDtypeStruct((_NSUP, _TSUP), jnp.float32),
        scratch_shapes=[pltpu.VMEM((_N, _FEAT), jnp.float32)],
        compiler_params=pltpu.CompilerParams(
            dimension_semantics=("arbitrary",),
        ),
    )(adj, x, wt, wb, b.reshape(1, _NHID), W1, b1.reshape(1, _NHID),
      alpha.reshape(1, _NHID), W2.reshape(1, _NHID), b2.reshape(1, 1))
    return out.reshape(-1)
